# Initial kernel scaffold; baseline (speedup 1.0000x reference)
import jax
import jax.numpy as jnp
from jax.experimental import pallas as pl


def kernel(amount, mcc, category, seq_lens):
    raise NotImplementedError("write your pallas kernel here")


# speedup vs baseline: 1.0773x; 1.0773x over previous
import functools

import jax
import jax.numpy as jnp
from jax import lax
from jax.experimental import pallas as pl
from jax.experimental.pallas import tpu as pltpu
from jax.experimental.pallas import tpu_sc as plsc

B, T = 1024, 200
KM, KC = 100, 32
KM_P = 112
KC_P = 32
L = 16
NC, NS = 2, 16
NW = NC * NS
RW = B // NW
NG = RW // L
EPS = 1e-9


def _sc_body(amt_hbm, mcc_hbm, cat_hbm,
             s_out, q_out, cm_out, sm_out, qm_out, cc_out, sc_out, qc_out,
             amt_v, mcc_v, cat_v,
             cm_v, sm_v, qm_v, cc_v, sc_v, qc_v, s_st, q_st):
    wid = lax.axis_index("s") * NC + lax.axis_index("c")
    base = wid * RW
    pltpu.sync_copy(amt_hbm.at[pl.ds(base, RW)], amt_v)
    pltpu.sync_copy(mcc_hbm.at[pl.ds(base, RW)], mcc_v)
    pltpu.sync_copy(cat_hbm.at[pl.ds(base, RW)], cat_v)

    zeros = jnp.zeros((L,), jnp.float32)
    ones = jnp.ones((L,), jnp.float32)
    lane = lax.iota(jnp.int32, L)

    for g in range(NG):
        for r in range(L):
            for ch in range(KM_P // L):
                cm_v[r, pl.ds(ch * L, L)] = zeros
                sm_v[r, pl.ds(ch * L, L)] = zeros
                qm_v[r, pl.ds(ch * L, L)] = zeros
            for ch in range(KC_P // L):
                cc_v[r, pl.ds(ch * L, L)] = zeros
                sc_v[r, pl.ds(ch * L, L)] = zeros
                qc_v[r, pl.ds(ch * L, L)] = zeros

        rows = lane + (g * L)

        def t_body(t, carry):
            s, q = carry
            t16 = jnp.full((L,), t, jnp.int32)
            a = plsc.load_gather(amt_v, [rows, t16])
            m = plsc.load_gather(mcc_v, [rows, t16])
            c = plsc.load_gather(cat_v, [rows, t16])
            v = (jnp.exp(jnp.abs(a)) - 1.0) * jnp.sign(a)
            v2 = v * v
            plsc.addupdate_scatter(cm_v, [lane, m], ones)
            plsc.addupdate_scatter(sm_v, [lane, m], v)
            plsc.addupdate_scatter(qm_v, [lane, m], v2)
            plsc.addupdate_scatter(cc_v, [lane, c], ones)
            plsc.addupdate_scatter(sc_v, [lane, c], v)
            plsc.addupdate_scatter(qc_v, [lane, c], v2)
            return s + v, q + v2

        s, q = lax.fori_loop(0, T, t_body, (zeros, zeros))
        s_st[...] = s
        q_st[...] = q

        gbase = base + g * L
        pltpu.sync_copy(s_st, s_out.at[pl.ds(gbase, L)])
        pltpu.sync_copy(q_st, q_out.at[pl.ds(gbase, L)])
        pltpu.sync_copy(cm_v, cm_out.at[pl.ds(gbase, L)])
        pltpu.sync_copy(sm_v, sm_out.at[pl.ds(gbase, L)])
        pltpu.sync_copy(qm_v, qm_out.at[pl.ds(gbase, L)])
        pltpu.sync_copy(cc_v, cc_out.at[pl.ds(gbase, L)])
        pltpu.sync_copy(sc_v, sc_out.at[pl.ds(gbase, L)])
        pltpu.sync_copy(qc_v, qc_out.at[pl.ds(gbase, L)])


def _sc_aggregate(amount, mcc, category):
    f32 = jnp.float32
    out_type = [
        jax.ShapeDtypeStruct((B,), f32),
        jax.ShapeDtypeStruct((B,), f32),
        jax.ShapeDtypeStruct((B, KM_P), f32),
        jax.ShapeDtypeStruct((B, KM_P), f32),
        jax.ShapeDtypeStruct((B, KM_P), f32),
        jax.ShapeDtypeStruct((B, KC_P), f32),
        jax.ShapeDtypeStruct((B, KC_P), f32),
        jax.ShapeDtypeStruct((B, KC_P), f32),
    ]
    scratch = [
        pltpu.VMEM((RW, T), f32),
        pltpu.VMEM((RW, T), jnp.int32),
        pltpu.VMEM((RW, T), jnp.int32),
        pltpu.VMEM((L, KM_P), f32),
        pltpu.VMEM((L, KM_P), f32),
        pltpu.VMEM((L, KM_P), f32),
        pltpu.VMEM((L, KC_P), f32),
        pltpu.VMEM((L, KC_P), f32),
        pltpu.VMEM((L, KC_P), f32),
        pltpu.VMEM((L,), f32),
        pltpu.VMEM((L,), f32),
    ]
    mesh = plsc.VectorSubcoreMesh(core_axis_name="c", subcore_axis_name="s",
                                  num_cores=NC, num_subcores=NS)
    fn = pl.kernel(_sc_body, out_type=out_type, mesh=mesh,
                   scratch_types=scratch,
                   compiler_params=pltpu.CompilerParams(
                       use_tc_tiling_on_sc=False,
                       needs_layout_passes=False))
    return fn(amount, mcc, category)


def _tc_finalize_body(sl_ref, s_ref, q_ref, cm_ref, sm_ref, qm_ref,
                      cc_ref, sc_ref, qc_ref, out_ref):
    sl = sl_ref[...]
    s = s_ref[...]
    q = q_ref[...]
    sl_e = sl + EPS
    mean_row = s / sl_e
    a_row = jnp.clip(q - (s * s) / sl_e, 0.0, None)
    std_row = jnp.sqrt(a_row / (jnp.clip(sl - 1.0, 0.0, None) + EPS))

    def block(cnt, sm, qm, k):
        kcols = lax.broadcasted_iota(jnp.int32, cnt.shape, 1)
        cnt_m = jnp.where(kcols == 0, 0.0, cnt)
        mean = sm / (cnt_m + EPS)
        a2 = jnp.clip(qm - (sm * sm) / (cnt_m + EPS), 0.0, None)
        std = jnp.sqrt(a2 / (jnp.clip(cnt_m - 1.0, 0.0, None) + EPS))
        ndist = jnp.sum((cnt_m > 0.0).astype(jnp.float32), axis=1,
                        keepdims=True)
        return cnt_m[:, :k], mean[:, :k], std[:, :k], ndist

    cm, mm, tm, nm = block(cm_ref[...], sm_ref[...], qm_ref[...], KM)
    cc, mc, tc, nc = block(cc_ref[...], sc_ref[...], qc_ref[...], KC)
    out_ref[...] = jnp.concatenate(
        [sl, s, mean_row, std_row, cm, mm, tm, cc, mc, tc, nm, nc], axis=1)


def _tc_finalize(sl, s, q, cm, sm, qm, cc, sc, qc):
    nfeat = 4 + 3 * KM + 3 * KC + 2
    return pl.pallas_call(
        _tc_finalize_body,
        out_shape=jax.ShapeDtypeStruct((B, nfeat), jnp.float32),
    )(sl, s, q, cm, sm, qm, cc, sc, qc)


def kernel(amount, mcc, category, seq_lens):
    amount = amount.astype(jnp.float32)
    mcc = mcc.astype(jnp.int32)
    category = category.astype(jnp.int32)
    s, q, cm, sm, qm, cc, sc, qc = _sc_aggregate(amount, mcc, category)
    sl = seq_lens.astype(jnp.float32).reshape(B, 1)
    return _tc_finalize(sl, s.reshape(B, 1), q.reshape(B, 1),
                        cm, sm, qm, cc, sc, qc)
